# speedup vs baseline: 14.5960x; 14.5960x over previous
import jax
import jax.numpy as jnp
from jax import lax
from jax.experimental import pallas as pl
from jax.experimental.pallas import tpu as pltpu
from jax.experimental.pallas import tpu_sc as plsc

_VOCAB = 100000
_D = 128
_B = 4096
_SEG = 200
_HID = 256
_NPRED = 1000

_NC, _NS = 2, 16
_NW = _NC * _NS
_RW = _B // _NW
_CH = 100
_CPR = _SEG // _CH
_NCHUNK = _RW * _CPR
_NB = 4


def _pool_body(x_hbm, table_hbm, out_hbm, idx_v, bufs, stage, *sems):
    c = lax.axis_index("c")
    s = lax.axis_index("s")
    w = c * _NS + s
    pltpu.sync_copy(x_hbm.at[w], idx_v)

    def gather(cidx, slot):
        return pltpu.make_async_copy(
            table_hbm.at[idx_v.at[cidx]], bufs.at[slot], sems[slot])

    for b in range(_NB):
        gather(b, b).start()

    def accum_chunk(slot, accs):
        def body(j, a):
            out = list(a)
            for u in range(4):
                row = j * 4 + u
                for k in range(8):
                    out[k] = out[k] + bufs[slot, row, pl.ds(k * 16, 16)]
            return tuple(out)
        return lax.fori_loop(0, _CH // 4, body, accs)

    def row_pair(i, issue_next):
        for rr in range(2):
            r = 2 * i + rr
            accs = tuple(jnp.zeros((16,), jnp.float32) for _ in range(8))
            for h in range(_CPR):
                slot = 2 * rr + h
                cidx = 4 * i + slot
                gather(cidx, slot).wait()
                accs = accum_chunk(slot, accs)
                if issue_next:
                    gather(cidx + _NB, slot).start()
            for k in range(8):
                stage[r, pl.ds(k * 16, 16)] = accs[k]

    def loop_body(i, carry):
        row_pair(i, True)
        return carry

    lax.fori_loop(0, _RW // 2 - 1, loop_body, 0)
    row_pair(_RW // 2 - 1, False)

    pltpu.sync_copy(stage, out_hbm.at[pl.ds(w * _RW, _RW)])


_pool = pl.kernel(
    _pool_body,
    out_type=jax.ShapeDtypeStruct((_B, _D), jnp.float32),
    mesh=plsc.VectorSubcoreMesh(
        core_axis_name="c", subcore_axis_name="s",
        num_cores=_NC, num_subcores=_NS),
    scratch_types=[
        pltpu.VMEM((_NCHUNK, _CH), jnp.int32),
        pltpu.VMEM((_NB, _CH, _D), jnp.float32),
        pltpu.VMEM((_RW, _D), jnp.float32),
    ] + [pltpu.SemaphoreType.DMA] * _NB,
)

_BB = 512
_NPAD = 1024


def _mlp_body(s_ref, w1_ref, b1_ref, w2_ref, b2_ref, out_ref):
    sv = s_ref[...]
    h = jnp.dot(sv, w1_ref[...], preferred_element_type=jnp.float32)
    h = h + b1_ref[...]
    h = 1.0 / (1.0 + jnp.exp(-h))
    logits = jnp.dot(h, w2_ref[...], preferred_element_type=jnp.float32)
    logits = logits + b2_ref[...]
    m = jnp.max(logits, axis=1, keepdims=True)
    lse = jnp.log(jnp.sum(jnp.exp(logits - m), axis=1, keepdims=True)) + m
    out_ref[...] = logits - lse


_mlp = pl.pallas_call(
    _mlp_body,
    grid=(_B // _BB,),
    in_specs=[
        pl.BlockSpec((_BB, _D), lambda i: (i, 0)),
        pl.BlockSpec((_D, _HID), lambda i: (0, 0)),
        pl.BlockSpec((1, _HID), lambda i: (0, 0)),
        pl.BlockSpec((_HID, _NPAD), lambda i: (0, 0)),
        pl.BlockSpec((1, _NPAD), lambda i: (0, 0)),
    ],
    out_specs=pl.BlockSpec((_BB, _NPAD), lambda i: (i, 0)),
    out_shape=jax.ShapeDtypeStruct((_B, _NPAD), jnp.float32),
)


def kernel(x, table, W1, b1, W2, b2):
    xr = x.reshape(_NW, _NCHUNK, _CH)
    s = _pool(xr, table)
    W2p = jnp.concatenate(
        [W2, jnp.zeros((_HID, _NPAD - _NPRED), W2.dtype)], axis=1)
    b2p = jnp.concatenate(
        [b2, jnp.full((_NPAD - _NPRED,), -1e30, b2.dtype)])
    out = _mlp(s, W1, b1.reshape(1, _HID), W2p, b2p.reshape(1, _NPAD))
    return out[:, :_NPRED]
